# baseline (speedup 1.0000x reference)
import functools

import jax
import jax.numpy as jnp
from jax import lax
from jax.experimental import pallas as pl
from jax.experimental.pallas import tpu as pltpu
from jax.experimental.pallas import tpu_sc as plsc

HIDDEN = 128
N_AA = 20
PEP_LENGTH = 15
MAX_TCR_LEN = 27
TOT_LEN = MAX_TCR_LEN + PEP_LENGTH
VOCAB = N_AA + 1
ENC_DIM = 32 + N_AA
VOC_PAD = 32
ENC_PAD = 128
G4 = 4 * HIDDEN
NB = 8


def _sc_gather(table, idx, n_rows):
    info = plsc.get_sparse_core_info()
    nw = info.num_cores * info.num_subcores
    b_per_w = n_rows // nw
    mesh = plsc.VectorSubcoreMesh(core_axis_name="c", subcore_axis_name="s")

    n_chunks = 2
    chunk = b_per_w // n_chunks

    @functools.partial(
        pl.kernel, mesh=mesh,
        out_type=jax.ShapeDtypeStruct((n_rows, ENC_PAD), jnp.float32),
        scratch_types=[
            pltpu.VMEM((chunk,), jnp.int32),
            pltpu.VMEM((chunk, ENC_PAD), jnp.float32),
            pltpu.SemaphoreType.DMA,
        ],
    )
    def gather_k(table_hbm, idx_hbm, out_hbm, idx_v, rows_v, sem):
        wid = lax.axis_index("s") * info.num_cores + lax.axis_index("c")
        for j in range(n_chunks):
            base = wid * b_per_w + j * chunk
            pltpu.sync_copy(idx_hbm.at[pl.ds(base, chunk)], idx_v)
            pltpu.async_copy(table_hbm.at[idx_v], rows_v, sem).wait()
            pltpu.sync_copy(rows_v, out_hbm.at[pl.ds(base, chunk)])

    return gather_k(table, idx)


def _sig(x):
    return 0.5 + 0.5 * jnp.tanh(0.5 * x)


def _cell(gates, c):
    i = _sig(gates[:, :HIDDEN])
    f = _sig(gates[:, HIDDEN:2 * HIDDEN])
    g = jnp.tanh(gates[:, 2 * HIDDEN:3 * HIDDEN])
    o = _sig(gates[:, 3 * HIDDEN:])
    c_new = f * c + i * g
    h_new = o * jnp.tanh(c_new)
    return h_new, c_new


def _dot(a, b):
    return jnp.dot(a, b, preferred_element_type=jnp.float32)


def _seq_kernel(x_ref, obs_ref,
                wih_pf_ref, wih_pb_ref, wih_tf_ref, wih_tb_ref,
                b_pf_ref, b_pb_ref, b_tf_ref, b_tb_ref,
                whh_pf_ref, whh_pb_ref, whh_tf_ref, whh_tb_ref,
                h0p_ref, c0p_ref, h0t_ref, c0t_ref,
                tcr_out_ref, tcr_hn_ref, pep_emb_ref):
    def wih(wih_ref, b_ref):
        pad = jnp.zeros((ENC_PAD - ENC_DIM - 1, G4), jnp.float32)
        return jnp.concatenate([wih_ref[...].T, b_ref[...], pad], axis=0)

    w_pf = wih(wih_pf_ref, b_pf_ref)
    w_pb = wih(wih_pb_ref, b_pb_ref)
    w_tf = wih(wih_tf_ref, b_tf_ref)
    w_tb = wih(wih_tb_ref, b_tb_ref)

    tcr_tok = obs_ref[:, :MAX_TCR_LEN]
    pep_tok = obs_ref[:, MAX_TCR_LEN:]
    lens_p = jnp.sum((pep_tok != 0).astype(jnp.int32), axis=1,
                     keepdims=True)
    lens_t = jnp.sum((tcr_tok != 0).astype(jnp.int32), axis=1, keepdims=True)

    def cell_step(x_t, wih_d, w, h, c, m):
        g = _dot(x_t, wih_d) + _dot(h, w)
        h_new, c_new = _cell(g, c)
        return jnp.where(m, h_new, h), jnp.where(m, c_new, c), h_new

    wpf, wpb = whh_pf_ref[...].T, whh_pb_ref[...].T
    wtf, wtb = whh_tf_ref[...].T, whh_tb_ref[...].T

    def tcr_step(i, hft, cft, hbt, cbt):
        tb = MAX_TCR_LEN - 1 - i
        mf = i < lens_t
        mb = tb < lens_t
        hft, cft, hf_new = cell_step(x_ref[i], w_tf, wtf, hft, cft, mf)
        hbt, cbt, hb_new = cell_step(x_ref[tb], w_tb, wtb, hbt, cbt, mb)
        tcr_out_ref[:, i, :HIDDEN] = jnp.where(mf, hf_new, 0.0)
        tcr_out_ref[:, tb, HIDDEN:] = jnp.where(mb, hb_new, 0.0)
        return hft, cft, hbt, cbt

    def body_a(i, carry):
        hfp, cfp, hbp, cbp, hft, cft, hbt, cbt = carry
        tb = PEP_LENGTH - 1 - i
        hfp, cfp, _ = cell_step(x_ref[MAX_TCR_LEN + i], w_pf, wpf, hfp, cfp,
                                i < lens_p)
        hbp, cbp, _ = cell_step(x_ref[MAX_TCR_LEN + tb], w_pb, wpb, hbp, cbp,
                                tb < lens_p)
        hft, cft, hbt, cbt = tcr_step(i, hft, cft, hbt, cbt)
        return hfp, cfp, hbp, cbp, hft, cft, hbt, cbt

    h0p, c0p = h0p_ref[...], c0p_ref[...]
    h0t, c0t = h0t_ref[...], c0t_ref[...]
    carry = (h0p, c0p, h0p, c0p, h0t, c0t, h0t, c0t)
    for i in range(PEP_LENGTH):
        carry = body_a(i, carry)
    hfp, _, hbp, _, hft, cft, hbt, cbt = carry
    carry4 = (hft, cft, hbt, cbt)
    for i in range(PEP_LENGTH, MAX_TCR_LEN):
        carry4 = tcr_step(i, *carry4)
    hft, _, hbt, _ = carry4

    pep_emb_ref[:, :HIDDEN] = hfp
    pep_emb_ref[:, HIDDEN:] = hbp
    tcr_hn_ref[0] = hft
    tcr_hn_ref[1] = hbt


@jax.jit
def kernel(obs, emb_table, onehot_dict, pep_Wih_f, pep_Whh_f, pep_b_f,
           pep_Wih_b, pep_Whh_b, pep_b_b, tcr_Wih_f, tcr_Whh_f, tcr_b_f,
           tcr_Wih_b, tcr_Whh_b, tcr_b_b, h0_pep, c0_pep, h0_tcr, c0_tcr):
    B = obs.shape[0]
    Bb = B // NB
    obs = obs.astype(jnp.int32)

    enc = jnp.zeros((VOC_PAD, ENC_PAD), jnp.float32)
    enc = enc.at[:VOCAB, :ENC_DIM].set(
        jnp.concatenate([emb_table, onehot_dict], axis=1))
    enc = enc.at[:VOCAB, ENC_DIM].set(1.0)

    idx_tm = obs.T.reshape(-1)
    x = _sc_gather(enc, idx_tm, TOT_LEN * B)
    x = x.reshape(TOT_LEN, B, ENC_PAD)

    args = (x, obs,
            pep_Wih_f, pep_Wih_b, tcr_Wih_f, tcr_Wih_b,
            pep_b_f.reshape(1, G4), pep_b_b.reshape(1, G4),
            tcr_b_f.reshape(1, G4), tcr_b_b.reshape(1, G4),
            pep_Whh_f, pep_Whh_b, tcr_Whh_f, tcr_Whh_b,
            h0_pep, c0_pep, h0_tcr, c0_tcr)

    full = lambda b: (0, 0)
    bat2 = lambda b: (b, 0)
    in_specs = [
        pl.BlockSpec((TOT_LEN, Bb, ENC_PAD), lambda b: (0, b, 0)),
        pl.BlockSpec((Bb, TOT_LEN), bat2),
        pl.BlockSpec((G4, ENC_DIM), full),
        pl.BlockSpec((G4, ENC_DIM), full),
        pl.BlockSpec((G4, ENC_DIM), full),
        pl.BlockSpec((G4, ENC_DIM), full),
        pl.BlockSpec((1, G4), full),
        pl.BlockSpec((1, G4), full),
        pl.BlockSpec((1, G4), full),
        pl.BlockSpec((1, G4), full),
        pl.BlockSpec((G4, HIDDEN), full),
        pl.BlockSpec((G4, HIDDEN), full),
        pl.BlockSpec((G4, HIDDEN), full),
        pl.BlockSpec((G4, HIDDEN), full),
        pl.BlockSpec((Bb, HIDDEN), bat2),
        pl.BlockSpec((Bb, HIDDEN), bat2),
        pl.BlockSpec((Bb, HIDDEN), bat2),
        pl.BlockSpec((Bb, HIDDEN), bat2),
    ]
    out_specs = [
        pl.BlockSpec((Bb, MAX_TCR_LEN, 2 * HIDDEN), lambda b: (b, 0, 0)),
        pl.BlockSpec((2, Bb, HIDDEN), lambda b: (0, b, 0)),
        pl.BlockSpec((Bb, 2 * HIDDEN), bat2),
    ]
    out_shapes = [
        jax.ShapeDtypeStruct((B, MAX_TCR_LEN, 2 * HIDDEN), jnp.float32),
        jax.ShapeDtypeStruct((2, B, HIDDEN), jnp.float32),
        jax.ShapeDtypeStruct((B, 2 * HIDDEN), jnp.float32),
    ]
    tcr_out, tcr_hn, pep_emb = pl.pallas_call(
        _seq_kernel,
        grid=(NB,),
        in_specs=in_specs,
        out_specs=out_specs,
        out_shape=out_shapes,
        compiler_params=pltpu.CompilerParams(
            dimension_semantics=("arbitrary",)),
    )(*args)
    return tcr_out, tcr_hn, pep_emb


# speedup vs baseline: 2.0741x; 2.0741x over previous
import jax
import jax.numpy as jnp
from jax.experimental import pallas as pl
from jax.experimental.pallas import tpu as pltpu

HIDDEN = 128
N_AA = 20
PEP_LENGTH = 15
MAX_TCR_LEN = 27
TOT_LEN = MAX_TCR_LEN + PEP_LENGTH
VOCAB = N_AA + 1
ENC_DIM = 32 + N_AA
VOC_PAD = 32
G4 = 4 * HIDDEN
NB = 8


def _sig(x):
    return 0.5 + 0.5 * jnp.tanh(0.5 * x)


def _cell(gates, c):
    i = _sig(gates[:, :HIDDEN])
    f = _sig(gates[:, HIDDEN:2 * HIDDEN])
    g = jnp.tanh(gates[:, 2 * HIDDEN:3 * HIDDEN])
    o = _sig(gates[:, 3 * HIDDEN:])
    c_new = f * c + i * g
    h_new = o * jnp.tanh(c_new)
    return h_new, c_new


def _dot(a, b):
    return jnp.dot(a, b, preferred_element_type=jnp.float32)


def _seq_kernel(obs_ref, emb_ref, onehot_ref,
                wih_pf_ref, wih_pb_ref, wih_tf_ref, wih_tb_ref,
                b_pf_ref, b_pb_ref, b_tf_ref, b_tb_ref,
                whh_pf_ref, whh_pb_ref, whh_tf_ref, whh_tb_ref,
                h0p_ref, c0p_ref, h0t_ref, c0t_ref,
                tcr_out_ref, tcr_hn_ref, pep_emb_ref,
                oh_pep_ref, oh_tcr_ref):
    enc = jnp.concatenate([emb_ref[...], onehot_ref[...]], axis=1)
    enc = jnp.pad(enc, ((0, VOC_PAD - VOCAB), (0, 0)))

    def tbl(wih_ref, b_ref):
        return _dot(enc, wih_ref[...].T) + b_ref[...]

    tbl_pf = tbl(wih_pf_ref, b_pf_ref)
    tbl_pb = tbl(wih_pb_ref, b_pb_ref)
    tbl_tf = tbl(wih_tf_ref, b_tf_ref)
    tbl_tb = tbl(wih_tb_ref, b_tb_ref)

    tcr_tok = obs_ref[:, :MAX_TCR_LEN]
    pep_tok = obs_ref[:, MAX_TCR_LEN:]

    Bb = obs_ref.shape[0]
    pep3 = pep_tok.T.reshape(PEP_LENGTH, Bb, 1)
    iota_p = jax.lax.broadcasted_iota(jnp.int32, (PEP_LENGTH, Bb, VOC_PAD), 2)
    oh_pep_ref[...] = (pep3 == iota_p).astype(jnp.float32)
    tcr3 = tcr_tok.T.reshape(MAX_TCR_LEN, Bb, 1)
    iota_t = jax.lax.broadcasted_iota(jnp.int32, (MAX_TCR_LEN, Bb, VOC_PAD), 2)
    oh_tcr_ref[...] = (tcr3 == iota_t).astype(jnp.float32)

    lens_p = jnp.sum((pep_tok != 0).astype(jnp.int32), axis=1,
                     keepdims=True)
    lens_t = jnp.sum((tcr_tok != 0).astype(jnp.int32), axis=1, keepdims=True)

    def cell_step(oh, tbl_d, w, h, c, m):
        g = _dot(oh, tbl_d) + _dot(h, w)
        h_new, c_new = _cell(g, c)
        return jnp.where(m, h_new, h), jnp.where(m, c_new, c), h_new

    wpf, wpb = whh_pf_ref[...].T, whh_pb_ref[...].T
    wtf, wtb = whh_tf_ref[...].T, whh_tb_ref[...].T

    def tcr_step(i, hft, cft, hbt, cbt):
        tb = MAX_TCR_LEN - 1 - i
        mf = i < lens_t
        mb = tb < lens_t
        hft, cft, hf_new = cell_step(oh_tcr_ref[i], tbl_tf, wtf, hft, cft, mf)
        hbt, cbt, hb_new = cell_step(oh_tcr_ref[tb], tbl_tb, wtb, hbt, cbt, mb)
        tcr_out_ref[:, i, :HIDDEN] = jnp.where(mf, hf_new, 0.0)
        tcr_out_ref[:, tb, HIDDEN:] = jnp.where(mb, hb_new, 0.0)
        return hft, cft, hbt, cbt

    def body_a(i, carry):
        hfp, cfp, hbp, cbp, hft, cft, hbt, cbt = carry
        tb = PEP_LENGTH - 1 - i
        hfp, cfp, _ = cell_step(oh_pep_ref[i], tbl_pf, wpf, hfp, cfp,
                                i < lens_p)
        hbp, cbp, _ = cell_step(oh_pep_ref[tb], tbl_pb, wpb, hbp, cbp,
                                tb < lens_p)
        hft, cft, hbt, cbt = tcr_step(i, hft, cft, hbt, cbt)
        return hfp, cfp, hbp, cbp, hft, cft, hbt, cbt

    h0p, c0p = h0p_ref[...], c0p_ref[...]
    h0t, c0t = h0t_ref[...], c0t_ref[...]
    carry = (h0p, c0p, h0p, c0p, h0t, c0t, h0t, c0t)
    for i in range(PEP_LENGTH):
        carry = body_a(i, carry)
    hfp, _, hbp, _, hft, cft, hbt, cbt = carry
    carry4 = (hft, cft, hbt, cbt)
    for i in range(PEP_LENGTH, MAX_TCR_LEN):
        carry4 = tcr_step(i, *carry4)
    hft, _, hbt, _ = carry4

    pep_emb_ref[:, :HIDDEN] = hfp
    pep_emb_ref[:, HIDDEN:] = hbp
    tcr_hn_ref[0] = hft
    tcr_hn_ref[1] = hbt


@jax.jit
def kernel(obs, emb_table, onehot_dict, pep_Wih_f, pep_Whh_f, pep_b_f,
           pep_Wih_b, pep_Whh_b, pep_b_b, tcr_Wih_f, tcr_Whh_f, tcr_b_f,
           tcr_Wih_b, tcr_Whh_b, tcr_b_b, h0_pep, c0_pep, h0_tcr, c0_tcr):
    B = obs.shape[0]
    Bb = B // NB

    args = (obs.astype(jnp.int32), emb_table, onehot_dict,
            pep_Wih_f, pep_Wih_b, tcr_Wih_f, tcr_Wih_b,
            pep_b_f.reshape(1, G4), pep_b_b.reshape(1, G4),
            tcr_b_f.reshape(1, G4), tcr_b_b.reshape(1, G4),
            pep_Whh_f, pep_Whh_b, tcr_Whh_f, tcr_Whh_b,
            h0_pep, c0_pep, h0_tcr, c0_tcr)

    full = lambda b: (0, 0)
    bat2 = lambda b: (b, 0)
    in_specs = [
        pl.BlockSpec((Bb, TOT_LEN), bat2),
        pl.BlockSpec((VOCAB, 32), full),
        pl.BlockSpec((VOCAB, N_AA), full),
        pl.BlockSpec((G4, ENC_DIM), full),
        pl.BlockSpec((G4, ENC_DIM), full),
        pl.BlockSpec((G4, ENC_DIM), full),
        pl.BlockSpec((G4, ENC_DIM), full),
        pl.BlockSpec((1, G4), full),
        pl.BlockSpec((1, G4), full),
        pl.BlockSpec((1, G4), full),
        pl.BlockSpec((1, G4), full),
        pl.BlockSpec((G4, HIDDEN), full),
        pl.BlockSpec((G4, HIDDEN), full),
        pl.BlockSpec((G4, HIDDEN), full),
        pl.BlockSpec((G4, HIDDEN), full),
        pl.BlockSpec((Bb, HIDDEN), bat2),
        pl.BlockSpec((Bb, HIDDEN), bat2),
        pl.BlockSpec((Bb, HIDDEN), bat2),
        pl.BlockSpec((Bb, HIDDEN), bat2),
    ]
    out_specs = [
        pl.BlockSpec((Bb, MAX_TCR_LEN, 2 * HIDDEN), lambda b: (b, 0, 0)),
        pl.BlockSpec((2, Bb, HIDDEN), lambda b: (0, b, 0)),
        pl.BlockSpec((Bb, 2 * HIDDEN), bat2),
    ]
    out_shapes = [
        jax.ShapeDtypeStruct((B, MAX_TCR_LEN, 2 * HIDDEN), jnp.float32),
        jax.ShapeDtypeStruct((2, B, HIDDEN), jnp.float32),
        jax.ShapeDtypeStruct((B, 2 * HIDDEN), jnp.float32),
    ]
    tcr_out, tcr_hn, pep_emb = pl.pallas_call(
        _seq_kernel,
        grid=(NB,),
        in_specs=in_specs,
        out_specs=out_specs,
        out_shape=out_shapes,
        scratch_shapes=[
            pltpu.VMEM((PEP_LENGTH, Bb, VOC_PAD), jnp.float32),
            pltpu.VMEM((MAX_TCR_LEN, Bb, VOC_PAD), jnp.float32),
        ],
        compiler_params=pltpu.CompilerParams(
            dimension_semantics=("parallel",)),
    )(*args)
    return tcr_out, tcr_hn, pep_emb
